# Initial kernel scaffold; baseline (speedup 1.0000x reference)
import jax
import jax.numpy as jnp
from jax.experimental import pallas as pl


def kernel(anchor, positive, negative):
    raise NotImplementedError("write your pallas kernel here")


# speedup vs baseline: 57.8725x; 57.8725x over previous
import functools

import jax
import jax.numpy as jnp
from jax import lax
from jax.experimental import pallas as pl
from jax.experimental.pallas import tpu as pltpu
from jax.experimental.pallas import tpu_sc as plsc

NC, NS = 2, 16
B, H, W = 16, 512, 512
NP = 255
G = 7225
NBLK = 85
L = NP * NP


def _tec_body(anchor, positive, negative, out_pos, out_neg,
              slab_a, slab_c, vbuf, rowtab, coltab, pjtab, rowbuf):
    wid = lax.axis_index("s") * NC + lax.axis_index("c")
    b = wid // 2
    comp = wid - 2 * b

    iota = lax.iota(jnp.int32, 16)

    def mk_tab_s(s, _):
        def mk_tab_j(j, _):
            lp = 144 * s + 9 * iota + j
            lp = jnp.minimum(lp, 764)
            lr = (lp >= 255).astype(jnp.int32) + (lp >= 510).astype(jnp.int32)
            lc = lp - 255 * lr
            off = (s * 9 + j) * 16
            rowtab[pl.ds(off, 16)] = 2 * lr
            coltab[pl.ds(off, 16)] = 2 * lc
            return 0
        return lax.fori_loop(0, 9, mk_tab_j, 0)
    lax.fori_loop(0, 6, mk_tab_s, 0)

    def mk_pj(k, _):
        x = 16 * k + iota
        pjtab[pl.ds(16 * k, 16)] = jnp.minimum(x // 2, 254)
        return 0
    lax.fori_loop(0, 32, mk_pj, 0)

    def block(t, _):
        @pl.when(comp == 0)
        def _():
            pltpu.sync_copy(positive.at[b, pl.ds(6 * t, 7), :], slab_c)

        @pl.when(comp == 1)
        def _():
            pltpu.sync_copy(negative.at[b, pl.ds(6 * t, 7), :], slab_c)

        pltpu.sync_copy(anchor.at[b, pl.ds(6 * t, 7), :], slab_a)

        def step(s, _):
            rows = [rowtab[pl.ds((s * 9 + j) * 16, 16)] for j in range(9)]
            cols = [coltab[pl.ds((s * 9 + j) * 16, 16)] for j in range(9)]
            mask = (16 * s + iota) < 85

            def chan(q, _):
                ki = (q >= 3).astype(jnp.int32) + (q >= 6).astype(jnp.int32)
                kj = q - 3 * ki
                dmin = dmax = cmin = cmax = None
                for j in range(9):
                    r = rows[j] + ki
                    cc = cols[j] + kj
                    a = plsc.load_gather(slab_a, [r, cc])
                    c = plsc.load_gather(slab_c, [r, cc])
                    d = jnp.abs(a - c)
                    if j == 0:
                        dmin = dmax = d
                        cmin = cmax = c
                    else:
                        lt = d < dmin
                        dmin = jnp.where(lt, d, dmin)
                        cmin = jnp.where(lt, c, cmin)
                        gt = d > dmax
                        dmax = jnp.where(gt, d, dmax)
                        cmax = jnp.where(gt, c, cmax)
                i0 = q * G + 85 * t + 16 * s
                plsc.store_scatter(vbuf, [i0 + iota], cmin + cmax, mask=mask)
                return 0

            return lax.fori_loop(0, 9, chan, 0)

        lax.fori_loop(0, 6, step, 0)
        return 0

    lax.fori_loop(0, NBLK, block, 0)

    def emit2(src, dst):
        pltpu.sync_copy(src, dst)

    def row(pi, _):
        def chunk(k, _):
            pj = pjtab[pl.ds(16 * k, 16)]
            v = plsc.load_gather(vbuf, [255 * pi + pj])
            v = jnp.where(16 * k + iota == 511, 0.0, v)
            rowbuf[0, pl.ds(16 * k, 16)] = v
            rowbuf[1, pl.ds(16 * k, 16)] = v
            return 0
        lax.fori_loop(0, 32, chunk, 0)

        @pl.when(comp == 0)
        def _():
            pltpu.sync_copy(rowbuf, out_pos.at[b, pl.ds(2 * pi, 2), :])

        @pl.when(comp == 1)
        def _():
            pltpu.sync_copy(rowbuf, out_neg.at[b, pl.ds(2 * pi, 2), :])

        @pl.when(pi == 254)
        def _():
            @pl.when(comp == 0)
            def _():
                pltpu.sync_copy(rowbuf.at[0], out_pos.at[b, 510])

            @pl.when(comp == 1)
            def _():
                pltpu.sync_copy(rowbuf.at[0], out_neg.at[b, 510])
        return 0

    lax.fori_loop(0, NP, row, 0)

    zeros = jnp.zeros((16,), jnp.float32)

    def zchunk(k, _):
        rowbuf[0, pl.ds(16 * k, 16)] = zeros
        return 0
    lax.fori_loop(0, 32, zchunk, 0)

    @pl.when(comp == 0)
    def _():
        pltpu.sync_copy(rowbuf.at[0], out_pos.at[b, 511])

    @pl.when(comp == 1)
    def _():
        pltpu.sync_copy(rowbuf.at[0], out_neg.at[b, 511])


@jax.jit
def _run(anchor, positive, negative):
    f = pl.kernel(
        _tec_body,
        out_type=(
            jax.ShapeDtypeStruct((B, H, W), jnp.float32),
            jax.ShapeDtypeStruct((B, H, W), jnp.float32),
        ),
        mesh=plsc.VectorSubcoreMesh(
            core_axis_name="c", subcore_axis_name="s",
            num_cores=NC, num_subcores=NS,
        ),
        scratch_types=[
            pltpu.VMEM((7, W), jnp.float32),
            pltpu.VMEM((7, W), jnp.float32),
            pltpu.VMEM((L,), jnp.float32),
            pltpu.VMEM((54 * 16,), jnp.int32),
            pltpu.VMEM((54 * 16,), jnp.int32),
            pltpu.VMEM((32 * 16,), jnp.int32),
            pltpu.VMEM((2, W), jnp.float32),
        ],
        compiler_params=pltpu.CompilerParams(use_tc_tiling_on_sc=False, needs_layout_passes=False),
    )
    return f(anchor, positive, negative)


def kernel(anchor, positive, negative):
    return _run(anchor, positive, negative)


# speedup vs baseline: 77.2120x; 1.3342x over previous
import functools

import jax
import jax.numpy as jnp
from jax import lax
from jax.experimental import pallas as pl
from jax.experimental.pallas import tpu as pltpu
from jax.experimental.pallas import tpu_sc as plsc

NC, NS = 2, 16
B, H, W = 16, 512, 512
NP = 255
G = 7225
NBLK = 85
L = NP * NP


def _tec_body(anchor, positive, negative, out_pos, out_neg,
              slab_a, slab_c, vbuf, rowtab, coltab, pjtab, rowbuf,
              sem_a, sem_c):
    wid = lax.axis_index("s") * NC + lax.axis_index("c")
    b = wid // 2
    comp = wid - 2 * b

    iota = lax.iota(jnp.int32, 16)

    def mk_tab_s(s, _):
        def mk_tab_j(j, _):
            lp = 144 * s + 9 * iota + j
            lp = jnp.minimum(lp, 764)
            lr = (lp >= 255).astype(jnp.int32) + (lp >= 510).astype(jnp.int32)
            lc = lp - 255 * lr
            off = (s * 9 + j) * 16
            rowtab[pl.ds(off, 16)] = 2 * lr
            coltab[pl.ds(off, 16)] = 2 * lc
            return 0
        return lax.fori_loop(0, 9, mk_tab_j, 0)
    lax.fori_loop(0, 6, mk_tab_s, 0)

    def mk_pj(k, _):
        x = 16 * k + iota
        pjtab[pl.ds(16 * k, 16)] = jnp.minimum(x // 2, 254)
        return 0
    lax.fori_loop(0, 32, mk_pj, 0)

    def start_fetch(t, buf):
        pltpu.async_copy(anchor.at[b, pl.ds(6 * t, 7), :],
                         slab_a.at[buf], sem_a.at[buf])

        @pl.when(comp == 0)
        def _():
            pltpu.async_copy(positive.at[b, pl.ds(6 * t, 7), :],
                             slab_c.at[buf], sem_c.at[buf])

        @pl.when(comp == 1)
        def _():
            pltpu.async_copy(negative.at[b, pl.ds(6 * t, 7), :],
                             slab_c.at[buf], sem_c.at[buf])

    def wait_fetch(buf):
        pltpu.make_async_copy(anchor.at[0, pl.ds(0, 7), :],
                              slab_a.at[buf], sem_a.at[buf]).wait()
        pltpu.make_async_copy(anchor.at[0, pl.ds(0, 7), :],
                              slab_c.at[buf], sem_c.at[buf]).wait()

    def compute_block(t, buf):
        sa = slab_a.at[buf]
        sc = slab_c.at[buf]

        def step(s, _):
            rows = [rowtab[pl.ds((s * 9 + j) * 16, 16)] for j in range(9)]
            cols = [coltab[pl.ds((s * 9 + j) * 16, 16)] for j in range(9)]
            mask = (16 * s + iota) < 85

            def chan(q, _):
                ki = (q >= 3).astype(jnp.int32) + (q >= 6).astype(jnp.int32)
                kj = q - 3 * ki
                dmin = dmax = cmin = cmax = None
                for j in range(9):
                    r = rows[j] + ki
                    cc = cols[j] + kj
                    a = plsc.load_gather(sa, [r, cc])
                    c = plsc.load_gather(sc, [r, cc])
                    d = jnp.abs(a - c)
                    if j == 0:
                        dmin = dmax = d
                        cmin = cmax = c
                    else:
                        lt = d < dmin
                        dmin = jnp.where(lt, d, dmin)
                        cmin = jnp.where(lt, c, cmin)
                        gt = d > dmax
                        dmax = jnp.where(gt, d, dmax)
                        cmax = jnp.where(gt, c, cmax)
                i0 = q * G + 85 * t + 16 * s
                plsc.store_scatter(vbuf, [i0 + iota], cmin + cmax, mask=mask)
                return 0

            return lax.fori_loop(0, 9, chan, 0)

        lax.fori_loop(0, 6, step, 0)

    start_fetch(0, 0)
    start_fetch(1, 1)

    def block_pair(tt, _):
        t = 2 * tt
        wait_fetch(0)
        compute_block(t, 0)
        start_fetch(t + 2, 0)
        wait_fetch(1)
        compute_block(t + 1, 1)

        @pl.when(tt < 41)
        def _():
            start_fetch(t + 3, 1)
        return 0

    lax.fori_loop(0, 42, block_pair, 0)
    wait_fetch(0)
    compute_block(84, 0)

    def emit2(src, dst):
        pltpu.sync_copy(src, dst)

    def row(pi, _):
        def chunk(k, _):
            pj = pjtab[pl.ds(16 * k, 16)]
            v = plsc.load_gather(vbuf, [255 * pi + pj])
            v = jnp.where(16 * k + iota == 511, 0.0, v)
            rowbuf[0, pl.ds(16 * k, 16)] = v
            rowbuf[1, pl.ds(16 * k, 16)] = v
            return 0
        lax.fori_loop(0, 32, chunk, 0)

        @pl.when(comp == 0)
        def _():
            pltpu.sync_copy(rowbuf, out_pos.at[b, pl.ds(2 * pi, 2), :])

        @pl.when(comp == 1)
        def _():
            pltpu.sync_copy(rowbuf, out_neg.at[b, pl.ds(2 * pi, 2), :])

        @pl.when(pi == 254)
        def _():
            @pl.when(comp == 0)
            def _():
                pltpu.sync_copy(rowbuf.at[0], out_pos.at[b, 510])

            @pl.when(comp == 1)
            def _():
                pltpu.sync_copy(rowbuf.at[0], out_neg.at[b, 510])
        return 0

    lax.fori_loop(0, NP, row, 0)

    zeros = jnp.zeros((16,), jnp.float32)

    def zchunk(k, _):
        rowbuf[0, pl.ds(16 * k, 16)] = zeros
        return 0
    lax.fori_loop(0, 32, zchunk, 0)

    @pl.when(comp == 0)
    def _():
        pltpu.sync_copy(rowbuf.at[0], out_pos.at[b, 511])

    @pl.when(comp == 1)
    def _():
        pltpu.sync_copy(rowbuf.at[0], out_neg.at[b, 511])


@jax.jit
def _run(anchor, positive, negative):
    f = pl.kernel(
        _tec_body,
        out_type=(
            jax.ShapeDtypeStruct((B, H, W), jnp.float32),
            jax.ShapeDtypeStruct((B, H, W), jnp.float32),
        ),
        mesh=plsc.VectorSubcoreMesh(
            core_axis_name="c", subcore_axis_name="s",
            num_cores=NC, num_subcores=NS,
        ),
        scratch_types=[
            pltpu.VMEM((2, 7, W), jnp.float32),
            pltpu.VMEM((2, 7, W), jnp.float32),
            pltpu.VMEM((L,), jnp.float32),
            pltpu.VMEM((54 * 16,), jnp.int32),
            pltpu.VMEM((54 * 16,), jnp.int32),
            pltpu.VMEM((32 * 16,), jnp.int32),
            pltpu.VMEM((2, W), jnp.float32),
            pltpu.SemaphoreType.DMA((2,)),
            pltpu.SemaphoreType.DMA((2,)),
        ],
        compiler_params=pltpu.CompilerParams(use_tc_tiling_on_sc=False, needs_layout_passes=False),
    )
    return f(anchor, positive, negative)


def kernel(anchor, positive, negative):
    return _run(anchor, positive, negative)


# speedup vs baseline: 96.5411x; 1.2503x over previous
import functools

import jax
import jax.numpy as jnp
from jax import lax
from jax.experimental import pallas as pl
from jax.experimental.pallas import tpu as pltpu
from jax.experimental.pallas import tpu_sc as plsc

NC, NS = 2, 16
B, H, W = 16, 512, 512
NP = 255
G = 7225
NBLK = 85
L = NP * NP


def _tec_body(anchor, positive, negative, out_pos, out_neg,
              slab_a, slab_c, vbuf, rowtab, coltab, pjtab, rowbuf, zbuf,
              sem_a, sem_c, sem_o, sem_z):
    wid = lax.axis_index("s") * NC + lax.axis_index("c")
    b = wid // 2
    comp = wid - 2 * b

    iota = lax.iota(jnp.int32, 16)

    def mk_tab_s(s, _):
        def mk_tab_j(j, _):
            lp = 144 * s + 9 * iota + j
            lp = jnp.minimum(lp, 764)
            lr = (lp >= 255).astype(jnp.int32) + (lp >= 510).astype(jnp.int32)
            lc = lp - 255 * lr
            off = (s * 9 + j) * 16
            rowtab[pl.ds(off, 16)] = 2 * lr
            coltab[pl.ds(off, 16)] = 2 * lc
            return 0
        return lax.fori_loop(0, 9, mk_tab_j, 0)
    lax.fori_loop(0, 6, mk_tab_s, 0)

    def mk_pj(k, _):
        x = 16 * k + iota
        pjtab[pl.ds(16 * k, 16)] = jnp.minimum(x // 2, 254)
        return 0
    lax.fori_loop(0, 32, mk_pj, 0)

    def start_fetch(t, buf):
        pltpu.async_copy(anchor.at[b, pl.ds(6 * t, 7), :],
                         slab_a.at[buf], sem_a.at[buf])

        @pl.when(comp == 0)
        def _():
            pltpu.async_copy(positive.at[b, pl.ds(6 * t, 7), :],
                             slab_c.at[buf], sem_c.at[buf])

        @pl.when(comp == 1)
        def _():
            pltpu.async_copy(negative.at[b, pl.ds(6 * t, 7), :],
                             slab_c.at[buf], sem_c.at[buf])

    def wait_fetch(buf):
        pltpu.make_async_copy(anchor.at[0, pl.ds(0, 7), :],
                              slab_a.at[buf], sem_a.at[buf]).wait()
        pltpu.make_async_copy(anchor.at[0, pl.ds(0, 7), :],
                              slab_c.at[buf], sem_c.at[buf]).wait()

    def compute_block(t, buf):
        sa = slab_a.at[buf]
        sc = slab_c.at[buf]

        def step(s, _):
            rows = [rowtab[pl.ds((s * 9 + j) * 16, 16)] for j in range(9)]
            cols = [coltab[pl.ds((s * 9 + j) * 16, 16)] for j in range(9)]
            mask = (16 * s + iota) < 85

            def chan(q, _):
                ki = (q >= 3).astype(jnp.int32) + (q >= 6).astype(jnp.int32)
                kj = q - 3 * ki
                dmin = dmax = cmin = cmax = None
                for j in range(9):
                    r = rows[j] + ki
                    cc = cols[j] + kj
                    a = plsc.load_gather(sa, [r, cc])
                    c = plsc.load_gather(sc, [r, cc])
                    d = jnp.abs(a - c)
                    if j == 0:
                        dmin = dmax = d
                        cmin = cmax = c
                    else:
                        lt = d < dmin
                        dmin = jnp.where(lt, d, dmin)
                        cmin = jnp.where(lt, c, cmin)
                        gt = d > dmax
                        dmax = jnp.where(gt, d, dmax)
                        cmax = jnp.where(gt, c, cmax)
                i0 = q * G + 85 * t + 16 * s
                plsc.store_scatter(vbuf, [i0 + iota], cmin + cmax, mask=mask)
                return 0

            return lax.fori_loop(0, 9, chan, 0)

        lax.fori_loop(0, 6, step, 0)

    start_fetch(0, 0)
    start_fetch(1, 1)

    def block_pair(tt, _):
        t = 2 * tt
        wait_fetch(0)
        compute_block(t, 0)
        start_fetch(t + 2, 0)
        wait_fetch(1)
        compute_block(t + 1, 1)

        @pl.when(tt < 41)
        def _():
            start_fetch(t + 3, 1)
        return 0

    lax.fori_loop(0, 42, block_pair, 0)
    wait_fetch(0)
    compute_block(84, 0)

    def build_row(pi, slot):
        base = 255 * pi
        for k in range(32):
            pj = pjtab[pl.ds(16 * k, 16)]
            v = plsc.load_gather(vbuf, [base + pj])
            if k == 31:
                v = jnp.where(iota == 15, 0.0, v)
            rowbuf[slot, 0, pl.ds(16 * k, 16)] = v
            rowbuf[slot, 1, pl.ds(16 * k, 16)] = v

    def issue_row(pi, slot):
        @pl.when(comp == 0)
        def _():
            pltpu.async_copy(rowbuf.at[slot],
                             out_pos.at[b, pl.ds(2 * pi, 2), :],
                             sem_o.at[slot])

        @pl.when(comp == 1)
        def _():
            pltpu.async_copy(rowbuf.at[slot],
                             out_neg.at[b, pl.ds(2 * pi, 2), :],
                             sem_o.at[slot])

    def wait_row(slot):
        pltpu.make_async_copy(rowbuf.at[slot],
                              out_pos.at[0, pl.ds(0, 2), :],
                              sem_o.at[slot]).wait()

    zeros = jnp.zeros((16,), jnp.float32)
    for k in range(32):
        zbuf[pl.ds(16 * k, 16)] = zeros

    @pl.when(comp == 0)
    def _():
        pltpu.async_copy(zbuf, out_pos.at[b, 511], sem_z)

    @pl.when(comp == 1)
    def _():
        pltpu.async_copy(zbuf, out_neg.at[b, 511], sem_z)

    def row4(rr, _):
        for po in range(4):
            pi = 4 * rr + po

            @pl.when(rr > 0)
            def _():
                wait_row(po)
            build_row(pi, po)
            issue_row(pi, po)
        return 0

    lax.fori_loop(0, 63, row4, 0)

    for po, pi in ((0, 252), (1, 253), (2, 254)):
        wait_row(po)
        build_row(pi, po)
        issue_row(pi, po)
    wait_row(3)

    @pl.when(comp == 0)
    def _():
        pltpu.async_copy(rowbuf.at[2, 0], out_pos.at[b, 510], sem_o.at[3])

    @pl.when(comp == 1)
    def _():
        pltpu.async_copy(rowbuf.at[2, 0], out_neg.at[b, 510], sem_o.at[3])

    for po in range(3):
        wait_row(po)
    pltpu.make_async_copy(zbuf, out_pos.at[0, 510], sem_o.at[3]).wait()
    pltpu.make_async_copy(zbuf, out_pos.at[0, 511], sem_z).wait()


@jax.jit
def _run(anchor, positive, negative):
    f = pl.kernel(
        _tec_body,
        out_type=(
            jax.ShapeDtypeStruct((B, H, W), jnp.float32),
            jax.ShapeDtypeStruct((B, H, W), jnp.float32),
        ),
        mesh=plsc.VectorSubcoreMesh(
            core_axis_name="c", subcore_axis_name="s",
            num_cores=NC, num_subcores=NS,
        ),
        scratch_types=[
            pltpu.VMEM((2, 7, W), jnp.float32),
            pltpu.VMEM((2, 7, W), jnp.float32),
            pltpu.VMEM((L,), jnp.float32),
            pltpu.VMEM((54 * 16,), jnp.int32),
            pltpu.VMEM((54 * 16,), jnp.int32),
            pltpu.VMEM((32 * 16,), jnp.int32),
            pltpu.VMEM((4, 2, W), jnp.float32),
            pltpu.VMEM((W,), jnp.float32),
            pltpu.SemaphoreType.DMA((2,)),
            pltpu.SemaphoreType.DMA((2,)),
            pltpu.SemaphoreType.DMA((4,)),
            pltpu.SemaphoreType.DMA,
        ],
        compiler_params=pltpu.CompilerParams(use_tc_tiling_on_sc=False, needs_layout_passes=False),
    )
    return f(anchor, positive, negative)


def kernel(anchor, positive, negative):
    return _run(anchor, positive, negative)


# speedup vs baseline: 100.1460x; 1.0373x over previous
import functools

import jax
import jax.numpy as jnp
from jax import lax
from jax.experimental import pallas as pl
from jax.experimental.pallas import tpu as pltpu
from jax.experimental.pallas import tpu_sc as plsc

NC, NS = 2, 16
B, H, W = 16, 512, 512
NP = 255
G = 7225
NBLK = 85
L = NP * NP


def _tec_body(anchor, positive, negative, out_pos, out_neg,
              slab_a, slab_c, vbuf, btab, pjtab, rowbuf, zbuf,
              sem_a, sem_c, sem_o, sem_z):
    wid = lax.axis_index("s") * NC + lax.axis_index("c")
    b = wid // 2
    comp = wid - 2 * b

    iota = lax.iota(jnp.int32, 16)

    def mk_tab_s(s, _):
        def mk_tab_j(j, _):
            lp = 144 * s + 9 * iota + j
            lp = jnp.minimum(lp, 764)
            lr = (lp >= 255).astype(jnp.int32) + (lp >= 510).astype(jnp.int32)
            lc = lp - 255 * lr
            btab[pl.ds((s * 9 + j) * 16, 16)] = 1024 * lr + 2 * lc
            return 0
        return lax.fori_loop(0, 9, mk_tab_j, 0)
    lax.fori_loop(0, 6, mk_tab_s, 0)

    def mk_pj(k, _):
        x = 16 * k + iota
        pjtab[pl.ds(16 * k, 16)] = jnp.minimum(x // 2, 254)
        return 0
    lax.fori_loop(0, 32, mk_pj, 0)

    SLAB = 7 * W

    def start_fetch(t, buf):
        for r in range(7):
            pltpu.async_copy(anchor.at[b, 6 * t + r, :],
                             slab_a.at[pl.ds(buf * SLAB + W * r, W)],
                             sem_a.at[buf])

        @pl.when(comp == 0)
        def _():
            for r in range(7):
                pltpu.async_copy(positive.at[b, 6 * t + r, :],
                                 slab_c.at[pl.ds(buf * SLAB + W * r, W)],
                                 sem_c.at[buf])

        @pl.when(comp == 1)
        def _():
            for r in range(7):
                pltpu.async_copy(negative.at[b, 6 * t + r, :],
                                 slab_c.at[pl.ds(buf * SLAB + W * r, W)],
                                 sem_c.at[buf])

    def wait_fetch(buf):
        pltpu.make_async_copy(anchor.at[0, pl.ds(0, 7), :],
                              slab_a.at[pl.ds(buf * SLAB, SLAB)],
                              sem_a.at[buf]).wait()
        pltpu.make_async_copy(anchor.at[0, pl.ds(0, 7), :],
                              slab_c.at[pl.ds(buf * SLAB, SLAB)],
                              sem_c.at[buf]).wait()

    def compute_block(t, buf):
        boff = buf * SLAB

        def step(s, _):
            bases = [btab[pl.ds((s * 9 + j) * 16, 16)] for j in range(9)]
            mask = (16 * s + iota) < 85

            def chan(q, _):
                ki = (q >= 3).astype(jnp.int32) + (q >= 6).astype(jnp.int32)
                kj = q - 3 * ki
                soff = 512 * ki + kj + boff
                dmin = dmax = cmin = cmax = None
                for j in range(9):
                    idx = bases[j] + soff
                    a = plsc.load_gather(slab_a, [idx])
                    c = plsc.load_gather(slab_c, [idx])
                    d = jnp.abs(a - c)
                    if j == 0:
                        dmin = dmax = d
                        cmin = cmax = c
                    else:
                        lt = d < dmin
                        dmin = jnp.where(lt, d, dmin)
                        cmin = jnp.where(lt, c, cmin)
                        gt = d > dmax
                        dmax = jnp.where(gt, d, dmax)
                        cmax = jnp.where(gt, c, cmax)
                i0 = q * G + 85 * t + 16 * s
                plsc.store_scatter(vbuf, [i0 + iota], cmin + cmax, mask=mask)
                return 0

            return lax.fori_loop(0, 9, chan, 0)

        lax.fori_loop(0, 6, step, 0)

    start_fetch(0, 0)
    start_fetch(1, 1)

    def block_pair(tt, _):
        t = 2 * tt
        wait_fetch(0)
        compute_block(t, 0)
        start_fetch(t + 2, 0)
        wait_fetch(1)
        compute_block(t + 1, 1)

        @pl.when(tt < 41)
        def _():
            start_fetch(t + 3, 1)
        return 0

    lax.fori_loop(0, 42, block_pair, 0)
    wait_fetch(0)
    compute_block(84, 0)

    def build_row(pi, slot):
        base = 255 * pi
        for k in range(32):
            pj = pjtab[pl.ds(16 * k, 16)]
            v = plsc.load_gather(vbuf, [base + pj])
            if k == 31:
                v = jnp.where(iota == 15, 0.0, v)
            rowbuf[slot, 0, pl.ds(16 * k, 16)] = v
            rowbuf[slot, 1, pl.ds(16 * k, 16)] = v

    def issue_row(pi, slot):
        @pl.when(comp == 0)
        def _():
            pltpu.async_copy(rowbuf.at[slot],
                             out_pos.at[b, pl.ds(2 * pi, 2), :],
                             sem_o.at[slot])

        @pl.when(comp == 1)
        def _():
            pltpu.async_copy(rowbuf.at[slot],
                             out_neg.at[b, pl.ds(2 * pi, 2), :],
                             sem_o.at[slot])

    def wait_row(slot):
        pltpu.make_async_copy(rowbuf.at[slot],
                              out_pos.at[0, pl.ds(0, 2), :],
                              sem_o.at[slot]).wait()

    zeros = jnp.zeros((16,), jnp.float32)
    for k in range(32):
        zbuf[pl.ds(16 * k, 16)] = zeros

    @pl.when(comp == 0)
    def _():
        pltpu.async_copy(zbuf, out_pos.at[b, 511], sem_z)

    @pl.when(comp == 1)
    def _():
        pltpu.async_copy(zbuf, out_neg.at[b, 511], sem_z)

    def row4(rr, _):
        for po in range(4):
            pi = 4 * rr + po

            @pl.when(rr > 0)
            def _():
                wait_row(po)
            build_row(pi, po)
            issue_row(pi, po)
        return 0

    lax.fori_loop(0, 63, row4, 0)

    for po, pi in ((0, 252), (1, 253), (2, 254)):
        wait_row(po)
        build_row(pi, po)
        issue_row(pi, po)
    wait_row(3)

    @pl.when(comp == 0)
    def _():
        pltpu.async_copy(rowbuf.at[2, 0], out_pos.at[b, 510], sem_o.at[3])

    @pl.when(comp == 1)
    def _():
        pltpu.async_copy(rowbuf.at[2, 0], out_neg.at[b, 510], sem_o.at[3])

    for po in range(3):
        wait_row(po)
    pltpu.make_async_copy(zbuf, out_pos.at[0, 510], sem_o.at[3]).wait()
    pltpu.make_async_copy(zbuf, out_pos.at[0, 511], sem_z).wait()


@jax.jit
def _run(anchor, positive, negative):
    f = pl.kernel(
        _tec_body,
        out_type=(
            jax.ShapeDtypeStruct((B, H, W), jnp.float32),
            jax.ShapeDtypeStruct((B, H, W), jnp.float32),
        ),
        mesh=plsc.VectorSubcoreMesh(
            core_axis_name="c", subcore_axis_name="s",
            num_cores=NC, num_subcores=NS,
        ),
        scratch_types=[
            pltpu.VMEM((2 * 7 * W,), jnp.float32),
            pltpu.VMEM((2 * 7 * W,), jnp.float32),
            pltpu.VMEM((L,), jnp.float32),
            pltpu.VMEM((54 * 16,), jnp.int32),
            pltpu.VMEM((32 * 16,), jnp.int32),
            pltpu.VMEM((4, 2, W), jnp.float32),
            pltpu.VMEM((W,), jnp.float32),
            pltpu.SemaphoreType.DMA((2,)),
            pltpu.SemaphoreType.DMA((2,)),
            pltpu.SemaphoreType.DMA((4,)),
            pltpu.SemaphoreType.DMA,
        ],
        compiler_params=pltpu.CompilerParams(use_tc_tiling_on_sc=False, needs_layout_passes=False),
    )
    return f(anchor, positive, negative)


def kernel(anchor, positive, negative):
    return _run(anchor, positive, negative)


# speedup vs baseline: 101.5698x; 1.0142x over previous
import functools

import jax
import jax.numpy as jnp
from jax import lax
from jax.experimental import pallas as pl
from jax.experimental.pallas import tpu as pltpu
from jax.experimental.pallas import tpu_sc as plsc

NC, NS = 2, 16
B, H, W = 16, 512, 512
NP = 255
G = 7225
NBLK = 85
L = NP * NP


def _tec_body(anchor, positive, negative, out_pos, out_neg,
              slab_a, slab_c, vbuf, btab, pjtab, rowbuf, zbuf,
              sem_a, sem_c, sem_o, sem_z):
    wid = lax.axis_index("s") * NC + lax.axis_index("c")
    b = wid // 2
    comp = wid - 2 * b

    iota = lax.iota(jnp.int32, 16)

    def mk_tab_s(s, _):
        def mk_tab_j(j, _):
            lp = 144 * s + 9 * iota + j
            lp = jnp.minimum(lp, 764)
            lr = (lp >= 255).astype(jnp.int32) + (lp >= 510).astype(jnp.int32)
            lc = lp - 255 * lr
            btab[pl.ds((s * 9 + j) * 16, 16)] = 1024 * lr + 2 * lc
            return 0
        return lax.fori_loop(0, 9, mk_tab_j, 0)
    lax.fori_loop(0, 6, mk_tab_s, 0)

    def mk_pj(k, _):
        x = 16 * k + iota
        pjtab[pl.ds(16 * k, 16)] = jnp.minimum(x // 2, 254)
        return 0
    lax.fori_loop(0, 32, mk_pj, 0)

    SLAB = 7 * W

    def start_fetch(t, buf):
        for r in range(7):
            pltpu.async_copy(anchor.at[b, 6 * t + r, :],
                             slab_a.at[pl.ds(buf * SLAB + W * r, W)],
                             sem_a.at[buf])

        @pl.when(comp == 0)
        def _():
            for r in range(7):
                pltpu.async_copy(positive.at[b, 6 * t + r, :],
                                 slab_c.at[pl.ds(buf * SLAB + W * r, W)],
                                 sem_c.at[buf])

        @pl.when(comp == 1)
        def _():
            for r in range(7):
                pltpu.async_copy(negative.at[b, 6 * t + r, :],
                                 slab_c.at[pl.ds(buf * SLAB + W * r, W)],
                                 sem_c.at[buf])

    def wait_fetch(buf):
        pltpu.make_async_copy(anchor.at[0, pl.ds(0, 7), :],
                              slab_a.at[pl.ds(buf * SLAB, SLAB)],
                              sem_a.at[buf]).wait()
        pltpu.make_async_copy(anchor.at[0, pl.ds(0, 7), :],
                              slab_c.at[pl.ds(buf * SLAB, SLAB)],
                              sem_c.at[buf]).wait()

    def compute_block(t, buf):
        boff = buf * SLAB

        def step(s, _):
            bases = [btab[pl.ds((s * 9 + j) * 16, 16)] for j in range(9)]
            mask = (16 * s + iota) < 85

            def chan(q, _):
                ki = (q >= 3).astype(jnp.int32) + (q >= 6).astype(jnp.int32)
                kj = q - 3 * ki
                soff = 512 * ki + kj + boff
                dmin = dmax = cmin = cmax = None
                for j in range(9):
                    idx = bases[j] + soff
                    a = plsc.load_gather(slab_a, [idx])
                    c = plsc.load_gather(slab_c, [idx])
                    d = jnp.abs(a - c)
                    if j == 0:
                        dmin = dmax = d
                        cmin = cmax = c
                    else:
                        lt = d < dmin
                        dmin = jnp.where(lt, d, dmin)
                        cmin = jnp.where(lt, c, cmin)
                        gt = d > dmax
                        dmax = jnp.where(gt, d, dmax)
                        cmax = jnp.where(gt, c, cmax)
                i0 = q * G + 85 * t + 16 * s
                plsc.store_scatter(vbuf, [i0 + iota], cmin + cmax, mask=mask)
                return 0

            return lax.fori_loop(0, 9, chan, 0)

        lax.fori_loop(0, 6, step, 0)

    start_fetch(0, 0)
    start_fetch(1, 1)

    def block_pair(tt, _):
        t = 2 * tt
        wait_fetch(0)
        compute_block(t, 0)
        start_fetch(t + 2, 0)
        wait_fetch(1)
        compute_block(t + 1, 1)

        @pl.when(tt < 41)
        def _():
            start_fetch(t + 3, 1)
        return 0

    lax.fori_loop(0, 42, block_pair, 0)
    wait_fetch(0)
    compute_block(84, 0)

    def build_row(pi, slot):
        base = 255 * pi
        for k in range(32):
            pj = pjtab[pl.ds(16 * k, 16)]
            v = plsc.load_gather(vbuf, [base + pj])
            if k == 31:
                v = jnp.where(iota == 15, 0.0, v)
            rowbuf[slot, pl.ds(16 * k, 16)] = v

    def issue_row(pi, slot):
        @pl.when(comp == 0)
        def _():
            pltpu.async_copy(rowbuf.at[slot], out_pos.at[b, 2 * pi],
                             sem_o.at[slot])
            pltpu.async_copy(rowbuf.at[slot], out_pos.at[b, 2 * pi + 1],
                             sem_o.at[slot])

        @pl.when(comp == 1)
        def _():
            pltpu.async_copy(rowbuf.at[slot], out_neg.at[b, 2 * pi],
                             sem_o.at[slot])
            pltpu.async_copy(rowbuf.at[slot], out_neg.at[b, 2 * pi + 1],
                             sem_o.at[slot])

    def wait_row(slot):
        pltpu.make_async_copy(rowbuf.at[slot], out_pos.at[0, 0],
                              sem_o.at[slot]).wait()
        pltpu.make_async_copy(rowbuf.at[slot], out_pos.at[0, 0],
                              sem_o.at[slot]).wait()

    zeros = jnp.zeros((16,), jnp.float32)
    for k in range(32):
        zbuf[pl.ds(16 * k, 16)] = zeros

    @pl.when(comp == 0)
    def _():
        pltpu.async_copy(zbuf, out_pos.at[b, 511], sem_z)

    @pl.when(comp == 1)
    def _():
        pltpu.async_copy(zbuf, out_neg.at[b, 511], sem_z)

    def row4(rr, _):
        for po in range(4):
            pi = 4 * rr + po

            @pl.when(rr > 0)
            def _():
                wait_row(po)
            build_row(pi, po)
            issue_row(pi, po)
        return 0

    lax.fori_loop(0, 63, row4, 0)

    for po, pi in ((0, 252), (1, 253), (2, 254)):
        wait_row(po)
        build_row(pi, po)
        issue_row(pi, po)
    wait_row(3)

    @pl.when(comp == 0)
    def _():
        pltpu.async_copy(rowbuf.at[2], out_pos.at[b, 510], sem_o.at[3])

    @pl.when(comp == 1)
    def _():
        pltpu.async_copy(rowbuf.at[2], out_neg.at[b, 510], sem_o.at[3])

    for po in range(3):
        wait_row(po)
    pltpu.make_async_copy(zbuf, out_pos.at[0, 510], sem_o.at[3]).wait()
    pltpu.make_async_copy(zbuf, out_pos.at[0, 511], sem_z).wait()


@jax.jit
def _run(anchor, positive, negative):
    f = pl.kernel(
        _tec_body,
        out_type=(
            jax.ShapeDtypeStruct((B, H, W), jnp.float32),
            jax.ShapeDtypeStruct((B, H, W), jnp.float32),
        ),
        mesh=plsc.VectorSubcoreMesh(
            core_axis_name="c", subcore_axis_name="s",
            num_cores=NC, num_subcores=NS,
        ),
        scratch_types=[
            pltpu.VMEM((2 * 7 * W,), jnp.float32),
            pltpu.VMEM((2 * 7 * W,), jnp.float32),
            pltpu.VMEM((L,), jnp.float32),
            pltpu.VMEM((54 * 16,), jnp.int32),
            pltpu.VMEM((32 * 16,), jnp.int32),
            pltpu.VMEM((4, W), jnp.float32),
            pltpu.VMEM((W,), jnp.float32),
            pltpu.SemaphoreType.DMA((2,)),
            pltpu.SemaphoreType.DMA((2,)),
            pltpu.SemaphoreType.DMA((4,)),
            pltpu.SemaphoreType.DMA,
        ],
        compiler_params=pltpu.CompilerParams(use_tc_tiling_on_sc=False, needs_layout_passes=False),
    )
    return f(anchor, positive, negative)


def kernel(anchor, positive, negative):
    return _run(anchor, positive, negative)


# speedup vs baseline: 106.8636x; 1.0521x over previous
import functools

import jax
import jax.numpy as jnp
from jax import lax
from jax.experimental import pallas as pl
from jax.experimental.pallas import tpu as pltpu
from jax.experimental.pallas import tpu_sc as plsc

NC, NS = 2, 16
B, H, W = 16, 512, 512
NP = 255
G = 7225
NBLK = 85
L = NP * NP


def _tec_body(anchor, positive, negative, out_pos, out_neg,
              slab_a, slab_c, vbuf, btab, pjtab, rowbuf, zbuf,
              sem_a, sem_c, sem_o, sem_z):
    wid = lax.axis_index("s") * NC + lax.axis_index("c")
    b = wid // 2
    comp = wid - 2 * b

    iota = lax.iota(jnp.int32, 16)

    def mk_tab_s(s, _):
        def mk_tab_j(j, _):
            lp = 144 * s + 9 * iota + j
            lp = jnp.minimum(lp, 764)
            lr = (lp >= 255).astype(jnp.int32) + (lp >= 510).astype(jnp.int32)
            lc = lp - 255 * lr
            btab[pl.ds((s * 9 + j) * 16, 16)] = 1024 * lr + 2 * lc
            return 0
        return lax.fori_loop(0, 9, mk_tab_j, 0)
    lax.fori_loop(0, 6, mk_tab_s, 0)

    def mk_pj(k, _):
        x = 16 * k + iota
        pjtab[pl.ds(16 * k, 16)] = jnp.minimum(x // 2, 254)
        return 0
    lax.fori_loop(0, 32, mk_pj, 0)

    SLAB = 7 * W

    def start_fetch(t, buf):
        for r in range(7):
            pltpu.async_copy(anchor.at[b, 6 * t + r, :],
                             slab_a.at[pl.ds(buf * SLAB + W * r, W)],
                             sem_a.at[buf])

        @pl.when(comp == 0)
        def _():
            for r in range(7):
                pltpu.async_copy(positive.at[b, 6 * t + r, :],
                                 slab_c.at[pl.ds(buf * SLAB + W * r, W)],
                                 sem_c.at[buf])

        @pl.when(comp == 1)
        def _():
            for r in range(7):
                pltpu.async_copy(negative.at[b, 6 * t + r, :],
                                 slab_c.at[pl.ds(buf * SLAB + W * r, W)],
                                 sem_c.at[buf])

    def wait_fetch(buf):
        pltpu.make_async_copy(anchor.at[0, pl.ds(0, 7), :],
                              slab_a.at[pl.ds(buf * SLAB, SLAB)],
                              sem_a.at[buf]).wait()
        pltpu.make_async_copy(anchor.at[0, pl.ds(0, 7), :],
                              slab_c.at[pl.ds(buf * SLAB, SLAB)],
                              sem_c.at[buf]).wait()

    def compute_block(t, buf):
        boff = buf * SLAB

        def step(s, _):
            bases = [btab[pl.ds((s * 9 + j) * 16, 16)] for j in range(9)]
            mask = (16 * s + iota) < 85

            def chan(q, _):
                ki = (q >= 3).astype(jnp.int32) + (q >= 6).astype(jnp.int32)
                kj = q - 3 * ki
                soff = 512 * ki + kj + boff
                dmin = dmax = cmin = cmax = None
                for j in range(9):
                    idx = bases[j] + soff
                    a = plsc.load_gather(slab_a, [idx])
                    c = plsc.load_gather(slab_c, [idx])
                    d = jnp.abs(a - c)
                    if j == 0:
                        dmin = dmax = d
                        cmin = cmax = c
                    else:
                        lt = d < dmin
                        dmin = jnp.where(lt, d, dmin)
                        cmin = jnp.where(lt, c, cmin)
                        gt = d > dmax
                        dmax = jnp.where(gt, d, dmax)
                        cmax = jnp.where(gt, c, cmax)
                i0 = q * G + 85 * t + 16 * s
                plsc.store_scatter(vbuf, [i0 + iota], cmin + cmax, mask=mask)
                return 0

            return lax.fori_loop(0, 9, chan, 0)

        lax.fori_loop(0, 6, step, 0)

    start_fetch(0, 0)
    start_fetch(1, 1)

    def block_pair(tt, _):
        t = 2 * tt
        wait_fetch(0)
        compute_block(t, 0)
        start_fetch(t + 2, 0)
        wait_fetch(1)
        compute_block(t + 1, 1)

        @pl.when(tt < 41)
        def _():
            start_fetch(t + 3, 1)
        return 0

    lax.fori_loop(0, 42, block_pair, 0)
    wait_fetch(0)
    compute_block(84, 0)

    def build_row(pi, slot):
        base = 255 * pi
        for k in range(32):
            pj = pjtab[pl.ds(16 * k, 16)]
            v = plsc.load_gather(vbuf, [base + pj])
            if k == 31:
                v = jnp.where(iota == 15, 0.0, v)
            rowbuf[slot, pl.ds(16 * k, 16)] = v

    def issue_row(pi, slot):
        @pl.when(comp == 0)
        def _():
            pltpu.async_copy(rowbuf.at[slot], out_pos.at[b, 2 * pi],
                             sem_o.at[slot])
            pltpu.async_copy(rowbuf.at[slot], out_pos.at[b, 2 * pi + 1],
                             sem_o.at[slot])

        @pl.when(comp == 1)
        def _():
            pltpu.async_copy(rowbuf.at[slot], out_neg.at[b, 2 * pi],
                             sem_o.at[slot])
            pltpu.async_copy(rowbuf.at[slot], out_neg.at[b, 2 * pi + 1],
                             sem_o.at[slot])

    def wait_row(slot):
        pltpu.make_async_copy(rowbuf.at[slot], out_pos.at[0, 0],
                              sem_o.at[slot]).wait()
        pltpu.make_async_copy(rowbuf.at[slot], out_pos.at[0, 0],
                              sem_o.at[slot]).wait()

    zeros = jnp.zeros((16,), jnp.float32)
    for k in range(32):
        zbuf[pl.ds(16 * k, 16)] = zeros

    @pl.when(comp == 0)
    def _():
        pltpu.async_copy(zbuf, out_pos.at[b, 511], sem_z)

    @pl.when(comp == 1)
    def _():
        pltpu.async_copy(zbuf, out_neg.at[b, 511], sem_z)

    def row4(rr, _):
        for po in range(4):
            pi = 4 * rr + po

            @pl.when(rr > 0)
            def _():
                wait_row(po)
            build_row(pi, po)
            issue_row(pi, po)
        return 0

    lax.fori_loop(0, 63, row4, 0)

    for po, pi in ((0, 252), (1, 253), (2, 254)):
        wait_row(po)
        build_row(pi, po)
        issue_row(pi, po)
    wait_row(3)

    @pl.when(comp == 0)
    def _():
        pltpu.async_copy(rowbuf.at[2], out_pos.at[b, 510], sem_o.at[3])

    @pl.when(comp == 1)
    def _():
        pltpu.async_copy(rowbuf.at[2], out_neg.at[b, 510], sem_o.at[3])

    for po in range(3):
        wait_row(po)
    pltpu.make_async_copy(zbuf, out_pos.at[0, 510], sem_o.at[3]).wait()
    pltpu.make_async_copy(zbuf, out_pos.at[0, 511], sem_z).wait()


from jax.experimental import layout as jax_layout

_LINEAR3 = jax_layout.Layout(major_to_minor=(0, 1, 2), tiling=())


@jax.jit
def _run(anchor, positive, negative):
    anchor = jax_layout.with_layout_constraint(anchor, _LINEAR3)
    positive = jax_layout.with_layout_constraint(positive, _LINEAR3)
    negative = jax_layout.with_layout_constraint(negative, _LINEAR3)
    f = pl.kernel(
        _tec_body,
        out_type=(
            jax.ShapeDtypeStruct((B, H, W), jnp.float32),
            jax.ShapeDtypeStruct((B, H, W), jnp.float32),
        ),
        mesh=plsc.VectorSubcoreMesh(
            core_axis_name="c", subcore_axis_name="s",
            num_cores=NC, num_subcores=NS,
        ),
        scratch_types=[
            pltpu.VMEM((2 * 7 * W,), jnp.float32),
            pltpu.VMEM((2 * 7 * W,), jnp.float32),
            pltpu.VMEM((L,), jnp.float32),
            pltpu.VMEM((54 * 16,), jnp.int32),
            pltpu.VMEM((32 * 16,), jnp.int32),
            pltpu.VMEM((4, W), jnp.float32),
            pltpu.VMEM((W,), jnp.float32),
            pltpu.SemaphoreType.DMA((2,)),
            pltpu.SemaphoreType.DMA((2,)),
            pltpu.SemaphoreType.DMA((4,)),
            pltpu.SemaphoreType.DMA,
        ],
        compiler_params=pltpu.CompilerParams(use_tc_tiling_on_sc=False, needs_layout_passes=False),
    )
    return f(anchor, positive, negative)


def kernel(anchor, positive, negative):
    return _run(anchor, positive, negative)
